# baseline (speedup 1.0000x reference)
import functools

import jax
import jax.numpy as jnp
from jax import lax
from jax.experimental import pallas as pl
from jax.experimental.pallas import tpu as pltpu
from jax.experimental.pallas import tpu_sc as plsc

VOCAB = 1000000
DIM = 64
BATCH = 16384
CTX = 20
NEG = 20
SCORE = 1 + NEG

NC = 2
NS = 16
NW = NC * NS
B_PER_W = BATCH // NW
R = 4
NCHUNK = B_PER_W // R
CTX_N = R * CTX
OUT_N = R * SCORE
NBUF = 4
FILL = 1.0e9


def _sc_dots(cat_tbl, ctx3d, outs3d):
    mesh = plsc.VectorSubcoreMesh(core_axis_name="c", subcore_axis_name="s")

    @functools.partial(
        pl.kernel,
        mesh=mesh,
        out_type=jax.ShapeDtypeStruct((BATCH, 32), jnp.float32),
        compiler_params=pltpu.CompilerParams(
            needs_layout_passes=False, use_tc_tiling_on_sc=False),
        scratch_types=[
            pltpu.VMEM((B_PER_W * CTX // CTX_N, CTX_N), jnp.int32),
            pltpu.VMEM((B_PER_W * SCORE // OUT_N, OUT_N), jnp.int32),
            pltpu.VMEM((NBUF, CTX_N, 2 * DIM), jnp.float32),
            pltpu.VMEM((NBUF, OUT_N, 2 * DIM), jnp.float32),
            pltpu.VMEM((NBUF, R, 32), jnp.float32),
            pltpu.SemaphoreType.DMA,
            pltpu.SemaphoreType.DMA,
            pltpu.SemaphoreType.DMA,
            pltpu.SemaphoreType.DMA,
            pltpu.SemaphoreType.DMA,
        ],
    )
    def k(tbl_hbm, ctx_hbm, outs_hbm, dots_o,
          ctx_idx, outs_idx, ctx_rows, outs_rows, dots_v,
          sem0, sem1, sem2, sem3, dsem):
        wid = lax.axis_index("s") * NC + lax.axis_index("c")
        lane = lax.iota(jnp.int32, 16)
        perms = [lane ^ (1 << p) for p in range(4)]
        sems = (sem0, sem1, sem2, sem3)

        def lanesum(x):
            for p in perms:
                x = x + jnp.take(x, p)
            return x
        pltpu.sync_copy(ctx_hbm.at[wid], ctx_idx)
        pltpu.sync_copy(outs_hbm.at[wid], outs_idx)

        def fire(i, buf):
            sem = sems[buf]
            pltpu.async_copy(tbl_hbm.at[ctx_idx.at[i]],
                             ctx_rows.at[buf], sem)
            pltpu.async_copy(tbl_hbm.at[outs_idx.at[i]],
                             outs_rows.at[buf], sem)

        def drain(buf):
            sem = sems[buf]
            pltpu.make_async_copy(tbl_hbm.at[pl.ds(0, CTX_N)],
                                  ctx_rows.at[buf], sem).wait()
            pltpu.make_async_copy(tbl_hbm.at[pl.ds(0, OUT_N)],
                                  outs_rows.at[buf], sem).wait()

        def ddrain():
            pltpu.make_async_copy(dots_o.at[pl.ds(0, R)],
                                  dots_v.at[0], dsem).wait()

        def compute(i, buf):
            @pl.when(i >= NBUF)
            def _():
                ddrain()

            def row_body(r, _):
                h = []
                for d in range(DIM // 16):
                    acc = ctx_rows[buf, r * CTX, pl.ds(d * 16, 16)]
                    for c in range(1, CTX):
                        acc = acc + ctx_rows[buf, r * CTX + c,
                                             pl.ds(d * 16, 16)]
                    h.append(acc * (1.0 / CTX))
                nh = [-v for v in h]
                v0 = jnp.full((16,), FILL)
                v1 = jnp.full((16,), FILL)
                for j in range(SCORE):
                    hh = h if j == 0 else nh
                    acc = outs_rows[buf, r * SCORE + j, pl.ds(DIM, 16)] * hh[0]
                    for d in range(1, DIM // 16):
                        acc = acc + outs_rows[buf, r * SCORE + j,
                                              pl.ds(DIM + d * 16, 16)] * hh[d]
                    dot = lanesum(acc)
                    if j < 16:
                        v0 = jnp.where(lane == j, dot, v0)
                    else:
                        v1 = jnp.where(lane == (j - 16), dot, v1)
                dots_v[buf, r, pl.ds(0, 16)] = v0
                dots_v[buf, r, pl.ds(16, 16)] = v1
                return 0

            lax.fori_loop(0, R, row_body, 0)
            pltpu.async_copy(dots_v.at[buf],
                             dots_o.at[pl.ds(wid * B_PER_W + i * R, R)],
                             dsem)

        for b in range(NBUF - 1):
            fire(b, b)

        def group_body(t, _):
            for p in range(NBUF):
                i = NBUF * t + p
                drain(p)
                compute(i, p)

                @pl.when(i + NBUF - 1 < NCHUNK)
                def _():
                    fire(i + NBUF - 1, (p + NBUF - 1) % NBUF)

            return 0

        lax.fori_loop(0, NCHUNK // NBUF, group_body, 0)
        for _ in range(NBUF):
            ddrain()

    return k(cat_tbl, ctx3d, outs3d)


def _tc_loss(dots2d):
    def body(dots_ref, out_ref):
        s = -jnp.sum(jax.nn.log_sigmoid(dots_ref[...])) / BATCH
        out_ref[...] = jnp.full((1, 1), s, dtype=jnp.float32)

    out = pl.pallas_call(
        body,
        out_shape=jax.ShapeDtypeStruct((1, 1), jnp.float32),
    )(dots2d)
    return out[0, 0]


def kernel(in_table, out_table, contexts, targets, negative_sampling):
    cat_tbl = jnp.concatenate([in_table, out_table], axis=1)
    ctx3d = contexts.astype(jnp.int32).reshape(
        NW, B_PER_W * CTX // CTX_N, CTX_N)
    outs = jnp.concatenate(
        [targets, negative_sampling], axis=1)
    outs3d = outs.astype(jnp.int32).reshape(
        NW, B_PER_W * SCORE // OUT_N, OUT_N)
    dots = _sc_dots(cat_tbl, ctx3d, outs3d)
    return _tc_loss(dots.reshape(BATCH * 32 // 128, 128))


# speedup vs baseline: 1.0107x; 1.0107x over previous
import functools

import jax
import jax.numpy as jnp
from jax import lax
from jax.experimental import pallas as pl
from jax.experimental.pallas import tpu as pltpu
from jax.experimental.pallas import tpu_sc as plsc

VOCAB = 1000000
DIM = 64
BATCH = 16384
CTX = 20
NEG = 20
SCORE = 1 + NEG

NC = 2
NS = 16
NW = NC * NS
B_PER_W = BATCH // NW
R = 4
NCHUNK = B_PER_W // R
CTX_N = R * CTX
OUT_N = R * SCORE
NBUF = 4
FILL = 1.0e9


def _sc_dots(cat_tbl, ctx3d, outs3d):
    mesh = plsc.VectorSubcoreMesh(core_axis_name="c", subcore_axis_name="s")

    @functools.partial(
        pl.kernel,
        mesh=mesh,
        out_type=jax.ShapeDtypeStruct((BATCH, 32), jnp.float32),
        compiler_params=pltpu.CompilerParams(
            needs_layout_passes=False, use_tc_tiling_on_sc=False),
        scratch_types=[
            pltpu.VMEM((B_PER_W * CTX // CTX_N, CTX_N), jnp.int32),
            pltpu.VMEM((B_PER_W * SCORE // OUT_N, OUT_N), jnp.int32),
            pltpu.VMEM((NBUF, CTX_N, 2 * DIM), jnp.float32),
            pltpu.VMEM((NBUF, OUT_N, 2 * DIM), jnp.float32),
            pltpu.VMEM((R, 32), jnp.float32),
            pltpu.SemaphoreType.DMA,
            pltpu.SemaphoreType.DMA,
            pltpu.SemaphoreType.DMA,
            pltpu.SemaphoreType.DMA,
        ],
    )
    def k(tbl_hbm, ctx_hbm, outs_hbm, dots_o,
          ctx_idx, outs_idx, ctx_rows, outs_rows, dots_v,
          sem0, sem1, sem2, sem3):
        wid = lax.axis_index("s") * NC + lax.axis_index("c")
        lane = lax.iota(jnp.int32, 16)
        perms = [lane ^ (1 << p) for p in range(4)]
        sems = (sem0, sem1, sem2, sem3)

        def lanesum(x):
            for p in perms:
                x = x + jnp.take(x, p)
            return x
        pltpu.sync_copy(ctx_hbm.at[wid], ctx_idx)
        pltpu.sync_copy(outs_hbm.at[wid], outs_idx)

        def fire(i, buf):
            sem = sems[buf]
            pltpu.async_copy(tbl_hbm.at[ctx_idx.at[i]],
                             ctx_rows.at[buf], sem)
            pltpu.async_copy(tbl_hbm.at[outs_idx.at[i]],
                             outs_rows.at[buf], sem)

        def drain(buf):
            sem = sems[buf]
            pltpu.make_async_copy(tbl_hbm.at[pl.ds(0, CTX_N)],
                                  ctx_rows.at[buf], sem).wait()
            pltpu.make_async_copy(tbl_hbm.at[pl.ds(0, OUT_N)],
                                  outs_rows.at[buf], sem).wait()

        def compute(i, buf):
            def row_body(r, _):
                h = []
                for d in range(DIM // 16):
                    acc = ctx_rows[buf, r * CTX, pl.ds(d * 16, 16)]
                    for c in range(1, CTX):
                        acc = acc + ctx_rows[buf, r * CTX + c,
                                             pl.ds(d * 16, 16)]
                    h.append(acc * (1.0 / CTX))
                nh = [-v for v in h]
                v0 = jnp.full((16,), FILL)
                v1 = jnp.full((16,), FILL)
                for j in range(SCORE):
                    hh = h if j == 0 else nh
                    acc = outs_rows[buf, r * SCORE + j, pl.ds(DIM, 16)] * hh[0]
                    for d in range(1, DIM // 16):
                        acc = acc + outs_rows[buf, r * SCORE + j,
                                              pl.ds(DIM + d * 16, 16)] * hh[d]
                    dot = lanesum(acc)
                    if j < 16:
                        v0 = jnp.where(lane == j, dot, v0)
                    else:
                        v1 = jnp.where(lane == (j - 16), dot, v1)
                dots_v[r, pl.ds(0, 16)] = v0
                dots_v[r, pl.ds(16, 16)] = v1
                return 0

            lax.fori_loop(0, R, row_body, 0)
            pltpu.sync_copy(dots_v,
                            dots_o.at[pl.ds(wid * B_PER_W + i * R, R)])

        for b in range(NBUF - 1):
            fire(b, b)

        def group_body(t, _):
            for p in range(NBUF):
                i = NBUF * t + p
                drain(p)
                compute(i, p)

                @pl.when(i + NBUF - 1 < NCHUNK)
                def _():
                    fire(i + NBUF - 1, (p + NBUF - 1) % NBUF)

            return 0

        lax.fori_loop(0, NCHUNK // NBUF, group_body, 0)

    return k(cat_tbl, ctx3d, outs3d)


def _tc_loss(dots2d):
    def body(dots_ref, out_ref):
        s = -jnp.sum(jax.nn.log_sigmoid(dots_ref[...])) / BATCH
        out_ref[...] = jnp.full((1, 1), s, dtype=jnp.float32)

    out = pl.pallas_call(
        body,
        out_shape=jax.ShapeDtypeStruct((1, 1), jnp.float32),
    )(dots2d)
    return out[0, 0]


def kernel(in_table, out_table, contexts, targets, negative_sampling):
    cat_tbl = jnp.concatenate([in_table, out_table], axis=1)
    ctx3d = contexts.astype(jnp.int32).reshape(
        NW, B_PER_W * CTX // CTX_N, CTX_N)
    outs = jnp.concatenate(
        [targets, negative_sampling], axis=1)
    outs3d = outs.astype(jnp.int32).reshape(
        NW, B_PER_W * SCORE // OUT_N, OUT_N)
    dots = _sc_dots(cat_tbl, ctx3d, outs3d)
    return _tc_loss(dots.reshape(BATCH * 32 // 128, 128))
